# baseline (speedup 1.0000x reference)
import functools

import jax
import jax.numpy as jnp
from jax import lax
from jax.experimental import pallas as pl
from jax.experimental.pallas import tpu as pltpu
from jax.experimental.pallas import tpu_sc as plsc

N = 10000
E = 160000
D_IN = 256
D_HID = 256
N_CLS = 64

NC = 2
NS = 16
L = 16

G = 96
NBLK = 108
EPT = NBLK * G
E_PAD = NS * EPT
NBLK_TOT = NS * NBLK
RPT = 624
REM = N - NS * RPT


def _make_spmm(DH):
  mesh = plsc.VectorSubcoreMesh(core_axis_name="c", subcore_axis_name="s")

  @functools.partial(
      pl.kernel,
      out_type=jax.ShapeDtypeStruct((NC, N, DH), jnp.float32),
      mesh=mesh,
      compiler_params=pltpu.CompilerParams(use_tc_tiling_on_sc=False),
      scratch_types=[
          pltpu.VMEM_SHARED((N, DH), jnp.float32),
          pltpu.VMEM((4, 2, G), jnp.int32),
          pltpu.VMEM((4, G), jnp.float32),
          pltpu.VMEM((4, G), jnp.int32),
          pltpu.VMEM((4, G, DH), jnp.float32),
          pltpu.SemaphoreType.DMA,
          pltpu.SemaphoreType.DMA,
          pltpu.SemaphoreType.DMA,
          pltpu.SemaphoreType.DMA,
      ],
  )
  def spmm(sup, pack, eww, out, acc, epb, ewb, ridx, rows,
           sem_e, sem_g, sem_s, sem_z):
    c = lax.axis_index("c")
    s = lax.axis_index("s")
    cn = c * N
    base = s * NBLK

    def load_edges(b, slot):
      pltpu.async_copy(pack.at[base + b], epb.at[slot], sem_e)
      pltpu.async_copy(eww.at[base + b], ewb.at[slot], sem_e)

    def wait_edges():
      pltpu.make_async_copy(pack.at[0], epb.at[0], sem_e).wait()
      pltpu.make_async_copy(eww.at[0], ewb.at[0], sem_e).wait()

    def adjust_col(slot):
      for j in range(G // L):
        sl = pl.ds(j * L, L)
        epb[slot, 1, sl] = epb[slot, 1, sl] + cn

    for b in range(3):
      load_edges(b, b)

    zero = jnp.zeros((L,), jnp.float32)

    def zrow(i, _):
      for r in range(4):
        for j in range(DH // L):
          rows[r, i, pl.ds(j * L, L)] = zero
      return 0

    lax.fori_loop(0, G, zrow, 0, unroll=False)
    zdescs = [
        pltpu.async_copy(rows.at[0], acc.at[pl.ds(s * RPT + k * G, G), :],
                         sem_z)
        for k in range(RPT // G)
    ]
    zdescs.append(
        pltpu.async_copy(rows.at[1, pl.ds(0, RPT - G * (RPT // G)), :],
                         acc.at[pl.ds(s * RPT + G * (RPT // G),
                                      RPT - G * (RPT // G)), :], sem_z))

    @pl.when(s == NS - 1)
    def _():
      pltpu.sync_copy(rows.at[2, pl.ds(0, REM), :],
                      acc.at[pl.ds(NS * RPT, REM), :])

    for d in zdescs:
      d.wait()
    plsc.subcore_barrier()

    wait_edges()
    wait_edges()
    adjust_col(0)
    adjust_col(1)
    pltpu.async_copy(sup.at[epb.at[0, 1]], rows.at[0], sem_g)
    pltpu.async_copy(sup.at[epb.at[1, 1]], rows.at[1], sem_g)
    pltpu.async_copy(rows.at[3], acc.at[epb.at[0, 0]], sem_s, add=True)
    pltpu.async_copy(rows.at[3], acc.at[epb.at[0, 0]], sem_s, add=True)

    def body(t, _):
      for pb in range(4):
        b = t * 4 + pb
        rA = rows.at[pb]
        pltpu.make_async_copy(sup.at[pl.ds(0, G)], rA, sem_g).wait()
        for j in range(G // L):
          sl = pl.ds(j * L, L)
          ridx[pb, sl] = epb[pb, 0, sl]

        def sgrp(gi, _):
          g0 = gi * L
          wv = ewb[pb, pl.ds(g0, L)]
          for l in range(L):
            w = wv[l]
            for j in range(DH // L):
              sl = pl.ds(j * L, L)
              rA[g0 + l, sl] = rA[g0 + l, sl] * w
          return 0

        lax.fori_loop(0, G // L, sgrp, 0, unroll=False)
        pltpu.make_async_copy(sup.at[pl.ds(0, G)], rA, sem_s).wait()
        pltpu.async_copy(rA, acc.at[ridx.at[pb]], sem_s, add=True)
        wait_edges()
        adjust_col((pb + 2) % 4)
        pltpu.async_copy(sup.at[epb.at[(pb + 2) % 4, 1]],
                         rows.at[(pb + 2) % 4], sem_g)
        load_edges(b + 3, (pb + 3) % 4)
      return 0

    lax.fori_loop(0, NBLK // 4, body, 0, unroll=False)
    pltpu.make_async_copy(sup.at[pl.ds(0, G)], rows.at[0], sem_g).wait()
    pltpu.make_async_copy(sup.at[pl.ds(0, G)], rows.at[1], sem_g).wait()
    pltpu.make_async_copy(sup.at[pl.ds(0, G)], rows.at[2], sem_s).wait()
    pltpu.make_async_copy(sup.at[pl.ds(0, G)], rows.at[3], sem_s).wait()
    wait_edges()
    plsc.subcore_barrier()

    pltpu.sync_copy(acc.at[pl.ds(s * RPT, RPT), :],
                    out.at[c, pl.ds(s * RPT, RPT), :])

    @pl.when(s == NS - 1)
    def _():
      pltpu.sync_copy(acc.at[pl.ds(NS * RPT, REM), :],
                      out.at[c, pl.ds(NS * RPT, REM), :])

  return spmm


_spmm_1 = _make_spmm(D_HID // NC)
_spmm_2 = _make_spmm(N_CLS // NC)


def _mm1_body(x_ref, w_ref, b_ref, o_ref):
  h = lax.dot_general(x_ref[...], w_ref[...], (((1,), (1,)), ((), ())),
                      preferred_element_type=jnp.float32)
  h = h + b_ref[...]
  o_ref[0] = h[:, :D_HID // 2]
  o_ref[1] = h[:, D_HID // 2:]


def _mm2_body(h_ref, wa_ref, wb_ref, b_ref, o_ref):
  a = jnp.maximum(h_ref[0], 0.0)
  b = jnp.maximum(h_ref[1], 0.0)
  s = lax.dot_general(a, wa_ref[...], (((1,), (1,)), ((), ())),
                      preferred_element_type=jnp.float32)
  s = s + lax.dot_general(b, wb_ref[...], (((1,), (1,)), ((), ())),
                          preferred_element_type=jnp.float32)
  s = s + b_ref[...]
  o_ref[0] = s[:, :N_CLS // 2]
  o_ref[1] = s[:, N_CLS // 2:]


_RB = 1000


def _mm1(x, W1, b1):
  return pl.pallas_call(
      _mm1_body,
      grid=(N // _RB,),
      in_specs=[
          pl.BlockSpec((_RB, D_IN), lambda i: (i, 0)),
          pl.BlockSpec((D_HID, D_IN), lambda i: (0, 0)),
          pl.BlockSpec((1, D_HID), lambda i: (0, 0)),
      ],
      out_specs=pl.BlockSpec((NC, _RB, D_HID // 2), lambda i: (0, i, 0)),
      out_shape=jax.ShapeDtypeStruct((NC, N, D_HID // 2), jnp.float32),
  )(x, W1, b1)


def _mm2(h, W2a, W2b, b2):
  return pl.pallas_call(
      _mm2_body,
      grid=(N // _RB,),
      in_specs=[
          pl.BlockSpec((NC, _RB, D_HID // 2), lambda i: (0, i, 0)),
          pl.BlockSpec((N_CLS, D_HID // 2), lambda i: (0, 0)),
          pl.BlockSpec((N_CLS, D_HID // 2), lambda i: (0, 0)),
          pl.BlockSpec((1, N_CLS), lambda i: (0, 0)),
      ],
      out_specs=pl.BlockSpec((NC, _RB, N_CLS // 2), lambda i: (0, i, 0)),
      out_shape=jax.ShapeDtypeStruct((NC, N, N_CLS // 2), jnp.float32),
  )(h, W2a, W2b, b2)


def _pack_edges(edge_index, edge_weight):
  pad = E_PAD - E
  rowp = jnp.pad(edge_index[0], (0, pad))
  colp = jnp.pad(edge_index[1], (0, pad))
  pack = jnp.stack([rowp, colp])
  pack = pack.reshape(2, NBLK_TOT, G).transpose(1, 0, 2)
  pack = jnp.pad(pack, ((0, 3), (0, 0), (0, 0)))
  eww = jnp.pad(edge_weight, (0, pad)).reshape(NBLK_TOT, G)
  eww = jnp.pad(eww, ((0, 3), (0, 0)))
  return pack, eww


def kernel(x, edge_index, edge_weight, W1, b1, W2, b2):
  pack, eww = _pack_edges(edge_index, edge_weight)
  sup1 = _mm1(x, W1, b1.reshape(1, D_HID))
  h = _spmm_1(sup1.reshape(NC * N, D_HID // 2), pack, eww)
  sup2 = _mm2(h, W2[:, :D_HID // 2], W2[:, D_HID // 2:],
              b2.reshape(1, N_CLS))
  o = _spmm_2(sup2.reshape(NC * N, N_CLS // 2), pack, eww)
  return jnp.concatenate([o[0], o[1]], axis=1)


# speedup vs baseline: 1.1835x; 1.1835x over previous
import functools

import jax
import jax.numpy as jnp
import numpy as np
from jax import lax
from jax.experimental import pallas as pl
from jax.experimental.pallas import tpu as pltpu
from jax.experimental.pallas import tpu_sc as plsc

N = 10000
E = 160000
D_IN = 256
D_HID = 256
N_CLS = 64

NC = 2
NS = 16
L = 16

G = 96
NBLK = 108
EPT = NBLK * G
E_PAD = NS * EPT
NBLK_TOT = NS * NBLK
RPT = 624
REM = N - NS * RPT


def _make_spmm(DH, bf16):
  mesh = plsc.VectorSubcoreMesh(core_axis_name="c", subcore_axis_name="s")
  tdt = jnp.bfloat16 if bf16 else jnp.float32
  n_rows = 3 if bf16 else 4

  @functools.partial(
      pl.kernel,
      out_type=jax.ShapeDtypeStruct((NC, N, DH), jnp.float32),
      mesh=mesh,
      compiler_params=pltpu.CompilerParams(use_tc_tiling_on_sc=False),
      scratch_types=[
          pltpu.VMEM_SHARED((N, DH), jnp.float32),
          pltpu.VMEM((4, 2, G), jnp.int32),
          pltpu.VMEM((4, G), jnp.float32),
          pltpu.VMEM((4, G), jnp.int32),
          (pltpu.VMEM((n_rows, G, DH // (2 * L), 2, L), tdt) if bf16
           else pltpu.VMEM((n_rows, G, DH), tdt)),
          pltpu.VMEM((2, G, DH) if bf16 else (1, 1), jnp.float32),
          pltpu.SemaphoreType.DMA,
          pltpu.SemaphoreType.DMA,
          pltpu.SemaphoreType.DMA,
          pltpu.SemaphoreType.DMA,
      ],
  )
  def spmm(sup, pack, eww, out, acc, epb, ewb, ridx, rows, stg,
           sem_e, sem_g, sem_s, sem_z):
    c = lax.axis_index("c")
    s = lax.axis_index("s")
    cn = c * N
    base = s * NBLK

    def load_edges(b, slot):
      pltpu.async_copy(pack.at[base + b], epb.at[slot], sem_e)
      pltpu.async_copy(eww.at[base + b], ewb.at[slot], sem_e)

    def wait_edges():
      pltpu.make_async_copy(pack.at[0], epb.at[0], sem_e).wait()
      pltpu.make_async_copy(eww.at[0], ewb.at[0], sem_e).wait()

    def adjust_col(slot):
      for j in range(G // L):
        sl = pl.ds(j * L, L)
        epb[slot, 1, sl] = epb[slot, 1, sl] + cn

    for b in range(3):
      load_edges(b, b)

    zero = jnp.zeros((L,), jnp.float32)
    zsrc = stg if bf16 else rows
    nz = 2 if bf16 else 4

    def zrow(i, _):
      for r in range(nz):
        for j in range(DH // L):
          zsrc[r, i, pl.ds(j * L, L)] = zero
      return 0

    lax.fori_loop(0, G, zrow, 0, unroll=False)
    zdescs = [
        pltpu.async_copy(zsrc.at[0], acc.at[pl.ds(s * RPT + k * G, G), :],
                         sem_z)
        for k in range(RPT // G)
    ]
    zdescs.append(
        pltpu.async_copy(zsrc.at[1, pl.ds(0, RPT - G * (RPT // G)), :],
                         acc.at[pl.ds(s * RPT + G * (RPT // G),
                                      RPT - G * (RPT // G)), :], sem_z))

    @pl.when(s == NS - 1)
    def _():
      pltpu.sync_copy(zsrc.at[0, pl.ds(0, REM), :],
                      acc.at[pl.ds(NS * RPT, REM), :])

    for d in zdescs:
      d.wait()
    plsc.subcore_barrier()

    wait_edges()
    wait_edges()
    adjust_col(0)
    adjust_col(1)
    pltpu.async_copy(sup.at[epb.at[0, 1]], rows.at[0], sem_g)
    pltpu.async_copy(sup.at[epb.at[1, 1]], rows.at[1], sem_g)

    def wait_gather(rb):
      pltpu.make_async_copy(sup.at[pl.ds(0, G)], rows.at[rb], sem_g).wait()

    def wait_scatter():
      if bf16:
        pltpu.make_async_copy(out.at[0, pl.ds(0, G), :], stg.at[0],
                              sem_s).wait()
      else:
        pltpu.make_async_copy(sup.at[pl.ds(0, G)], rows.at[0], sem_s).wait()

    def stash_ridx(slot):
      for j in range(G // L):
        sl = pl.ds(j * L, L)
        ridx[slot, sl] = epb[slot, 0, sl]

    if bf16:
      pltpu.async_copy(stg.at[1], acc.at[epb.at[0, 0]], sem_s, add=True)

      def body(t, _):
        for pb in range(12):
          b = t * 12 + pb
          rb, sg, eb = pb % 3, pb % 2, pb % 4
          wait_gather(rb)
          stash_ridx(eb)

          def sgrp(gi, _):
            g0 = gi * L
            wv = ewb[eb, pl.ds(g0, L)]
            for l in range(L):
              w = wv[l]
              r = g0 + l
              for j in range(DH // (2 * L)):
                v = rows[rb, r, j, :, :]
                vf = v.astype(jnp.float32)
                stg[sg, r, pl.ds(j * 2 * L, L)] = vf[0] * w
                stg[sg, r, pl.ds(j * 2 * L + L, L)] = vf[1] * w
            return 0

          lax.fori_loop(0, G // L, sgrp, 0, unroll=False)
          wait_scatter()
          pltpu.async_copy(stg.at[sg], acc.at[ridx.at[eb]], sem_s, add=True)
          wait_edges()
          adjust_col((pb + 2) % 4)
          pltpu.async_copy(sup.at[epb.at[(pb + 2) % 4, 1]],
                           rows.at[(pb + 2) % 3], sem_g)
          load_edges(b + 3, (pb + 3) % 4)
        return 0

      lax.fori_loop(0, NBLK // 12, body, 0, unroll=False)
      n_scat_drain = 1
    else:
      pltpu.async_copy(rows.at[3], acc.at[epb.at[0, 0]], sem_s, add=True)
      pltpu.async_copy(rows.at[3], acc.at[epb.at[0, 0]], sem_s, add=True)

      def body(t, _):
        for pb in range(4):
          b = t * 4 + pb
          rA = rows.at[pb]
          wait_gather(pb)
          stash_ridx(pb)

          def sgrp(gi, _):
            g0 = gi * L
            wv = ewb[pb, pl.ds(g0, L)]
            for l in range(L):
              w = wv[l]
              for j in range(DH // L):
                sl = pl.ds(j * L, L)
                rA[g0 + l, sl] = rA[g0 + l, sl] * w
            return 0

          lax.fori_loop(0, G // L, sgrp, 0, unroll=False)
          wait_scatter()
          pltpu.async_copy(rA, acc.at[ridx.at[pb]], sem_s, add=True)
          wait_edges()
          adjust_col((pb + 2) % 4)
          pltpu.async_copy(sup.at[epb.at[(pb + 2) % 4, 1]],
                           rows.at[(pb + 2) % 4], sem_g)
          load_edges(b + 3, (pb + 3) % 4)
        return 0

      lax.fori_loop(0, NBLK // 4, body, 0, unroll=False)
      n_scat_drain = 2

    wait_gather(0)
    wait_gather(1)
    for _ in range(n_scat_drain):
      wait_scatter()
    wait_edges()
    plsc.subcore_barrier()

    pltpu.sync_copy(acc.at[pl.ds(s * RPT, RPT), :],
                    out.at[c, pl.ds(s * RPT, RPT), :])

    @pl.when(s == NS - 1)
    def _():
      pltpu.sync_copy(acc.at[pl.ds(NS * RPT, REM), :],
                      out.at[c, pl.ds(NS * RPT, REM), :])

  return spmm


_spmm_1 = _make_spmm(D_HID // NC, True)
_spmm_2 = _make_spmm(N_CLS // NC, False)

def _mm1_body(x_ref, w_ref, b_ref, o_ref):
  h = lax.dot_general(x_ref[...], w_ref[...], (((1,), (1,)), ((), ())),
                      preferred_element_type=jnp.float32)
  h = h + b_ref[...]
  o_ref[0] = h[:, :D_HID // 2].astype(jnp.bfloat16)
  o_ref[1] = h[:, D_HID // 2:].astype(jnp.bfloat16)


def _mm2_body(h_ref, wa_ref, wb_ref, b_ref, o_ref):
  a = jnp.maximum(h_ref[0], 0.0)
  b = jnp.maximum(h_ref[1], 0.0)
  s = lax.dot_general(a, wa_ref[...], (((1,), (1,)), ((), ())),
                      preferred_element_type=jnp.float32)
  s = s + lax.dot_general(b, wb_ref[...], (((1,), (1,)), ((), ())),
                          preferred_element_type=jnp.float32)
  s = s + b_ref[...]
  o_ref[0] = s[:, :N_CLS // 2]
  o_ref[1] = s[:, N_CLS // 2:]


_RB = 2000


def _mm1(x, W1, b1):
  return pl.pallas_call(
      _mm1_body,
      grid=(N // _RB,),
      in_specs=[
          pl.BlockSpec((_RB, D_IN), lambda i: (i, 0)),
          pl.BlockSpec((D_HID, D_IN), lambda i: (0, 0)),
          pl.BlockSpec((1, D_HID), lambda i: (0, 0)),
      ],
      out_specs=pl.BlockSpec((NC, _RB, D_HID // 2), lambda i: (0, i, 0)),
      out_shape=jax.ShapeDtypeStruct((NC, N, D_HID // 2), jnp.bfloat16),
  )(x, W1, b1)


def _mm2(h, W2a, W2b, b2):
  return pl.pallas_call(
      _mm2_body,
      grid=(N // _RB,),
      in_specs=[
          pl.BlockSpec((NC, _RB, D_HID // 2), lambda i: (0, i, 0)),
          pl.BlockSpec((N_CLS, D_HID // 2), lambda i: (0, 0)),
          pl.BlockSpec((N_CLS, D_HID // 2), lambda i: (0, 0)),
          pl.BlockSpec((1, N_CLS), lambda i: (0, 0)),
      ],
      out_specs=pl.BlockSpec((NC, _RB, N_CLS // 2), lambda i: (0, i, 0)),
      out_shape=jax.ShapeDtypeStruct((NC, N, N_CLS // 2), jnp.float32),
  )(h, W2a, W2b, b2)


def _pack_edges(edge_index, edge_weight):
  pad = E_PAD - E
  rowp = jnp.pad(edge_index[0], (0, pad))
  colp = jnp.pad(edge_index[1], (0, pad))
  pack = jnp.stack([rowp, colp])
  pack = pack.reshape(2, NBLK_TOT, G).transpose(1, 0, 2)
  pack = jnp.pad(pack, ((0, 3), (0, 0), (0, 0)))
  eww = jnp.pad(edge_weight, (0, pad)).reshape(NBLK_TOT, G)
  eww = jnp.pad(eww, ((0, 3), (0, 0)))
  return pack, eww


def kernel(x, edge_index, edge_weight, W1, b1, W2, b2):
  pack, eww = _pack_edges(edge_index, edge_weight)
  sup1 = _mm1(x, W1, b1.reshape(1, D_HID))
  sup1 = sup1.reshape(NC * N, D_HID // 2 // (2 * 16), 2, 16)
  h = _spmm_1(sup1, pack, eww)
  sup2 = _mm2(h, W2[:, :D_HID // 2], W2[:, D_HID // 2:],
              b2.reshape(1, N_CLS))
  o = _spmm_2(sup2.reshape(NC * N, N_CLS // 2), pack, eww)
  return jnp.concatenate([o[0], o[1]], axis=1)
